# baseline (speedup 1.0000x reference)
import functools

import jax
import jax.numpy as jnp
from jax import lax
from jax.experimental import pallas as pl
from jax.experimental.pallas import tpu as pltpu
from jax.experimental.pallas import tpu_sc as plsc

H = 512
W = 512
B = 8
N = 4096
HW = H * W

NC = 2
NS = 16
NW = NC * NS
SPW = (B * N) // NW
CHUNKS = SPW // 16
WPB = NW // B

OPLANE = (N // 8) * 8 * 128
OUT_WORDS = B * OPLANE

_MESH = plsc.VectorSubcoreMesh(core_axis_name="c", subcore_axis_name="s")


@functools.partial(
    pl.kernel,
    mesh=_MESH,
    out_type=[
        jax.ShapeDtypeStruct((OUT_WORDS,), jnp.int32),
        jax.ShapeDtypeStruct((OUT_WORDS,), jnp.float32),
        jax.ShapeDtypeStruct((OUT_WORDS,), jnp.float32),
    ],
    scratch_types=[
        pltpu.VMEM((SPW,), jnp.int32),
        pltpu.VMEM((SPW * 2,), jnp.int32),
        pltpu.VMEM((SPW * 2,), jnp.int32),
        pltpu.VMEM((SPW * 3,), jnp.int32),
        pltpu.VMEM((SPW * 3,), jnp.int32),
        pltpu.VMEM((SPW * 3,), jnp.float32),
        pltpu.VMEM((SPW * 3,), jnp.float32),
        pltpu.SemaphoreType.DMA,
        pltpu.SemaphoreType.DMA,
        pltpu.SemaphoreType.DMA,
        pltpu.SemaphoreType.DMA,
        pltpu.SemaphoreType.DMA,
    ],
    compiler_params=pltpu.CompilerParams(needs_layout_passes=False),
)
def _sample_gather(idx_hbm, dirs_hbm, orig_hbm,
                   coords_out, dirs_out, orig_out,
                   idx_v, coords_v, cidx_v, fidx_v, sidx_v, dgat_v, ogat_v,
                   dsem, osem, csem, dssem, ossem):
    wid = lax.axis_index("s") * NC + lax.axis_index("c")
    b = wid // WPB
    pltpu.sync_copy(idx_hbm.at[pl.ds(wid * SPW, SPW)], idx_v)

    iota = lax.iota(jnp.int32, 16)
    out_tile_off = ((iota >> 3) << 10) + ((iota & 7) << 7)
    obase = b * OPLANE + (wid % WPB) * (SPW // 8) * 1024

    def body(j, carry):
        p0 = j * 16
        v = idx_v[pl.ds(p0, 16)]
        y = v >> 9
        x = v & 511
        qy = 2 * p0 + 2 * iota
        plsc.store_scatter(coords_v, [qy], y)
        plsc.store_scatter(coords_v, [qy + 1], x)
        toff = (((y >> 3) << 12) + ((x >> 7) << 10)
                + ((y & 7) << 7) + (x & 127))
        vob = (obase + ((p0 >> 3) << 10)) + out_tile_off
        plsc.store_scatter(cidx_v, [qy], vob)
        plsc.store_scatter(cidx_v, [qy + 1], vob + 1)
        q0 = 3 * p0 + 3 * iota
        for c in range(3):
            plsc.store_scatter(fidx_v, [q0 + c], toff + ((b * 3 + c) * HW))
            plsc.store_scatter(sidx_v, [q0 + c], vob + c)
        return carry

    lax.fori_loop(0, CHUNKS, body, 0)

    ccp = pltpu.async_copy(coords_v, coords_out.at[cidx_v], csem)
    dcp = pltpu.async_copy(dirs_hbm.at[fidx_v], dgat_v, dsem)
    ocp = pltpu.async_copy(orig_hbm.at[fidx_v], ogat_v, osem)
    dcp.wait()
    dscp = pltpu.async_copy(dgat_v, dirs_out.at[sidx_v], dssem)
    ocp.wait()
    oscp = pltpu.async_copy(ogat_v, orig_out.at[sidx_v], ossem)
    ccp.wait()
    dscp.wait()
    oscp.wait()


def kernel(n_sample, rays_directions, rays_origins):
    indices = jax.random.randint(jax.random.key(42), (B, N), 0, HW)
    idx_flat = indices.reshape(-1).astype(jnp.int32)

    def tiled_flat(a):
        return (a.reshape(B, 3, H // 8, 8, W // 128, 128)
                 .transpose(0, 1, 2, 4, 3, 5)
                 .reshape(-1))

    coords_pad, dirs_pad, orig_pad = _sample_gather(
        idx_flat,
        tiled_flat(rays_directions),
        tiled_flat(rays_origins),
    )

    sample_coordinates = coords_pad.reshape(B, N, 128)[:, :, :2]
    sampled_dirs = dirs_pad.reshape(B, N, 128)[:, :, :3]
    sampled_origins = orig_pad.reshape(B, N, 128)[:, :, :3]
    indices = indices + (jnp.asarray(n_sample, dtype=indices.dtype) * 0)
    return indices, sample_coordinates, sampled_dirs, sampled_origins


# speedup vs baseline: 4.1124x; 4.1124x over previous
import functools

import jax
import jax.numpy as jnp
from jax import lax
from jax.experimental import pallas as pl
from jax.experimental.pallas import tpu as pltpu
from jax.experimental.pallas import tpu_sc as plsc

H = 512
W = 512
B = 8
N = 4096
HW = H * W

NC = 2
NS = 16
NW = NC * NS
SPW = (B * N) // NW
CHUNKS = SPW // 16
WPB = NW // B

OPLANE = (N // 8) * 8 * 128
OUT_WORDS = B * OPLANE

QSAMP = SPW // 4
QWORDS = (QSAMP // 8) * 1024

_MESH = plsc.VectorSubcoreMesh(core_axis_name="c", subcore_axis_name="s")


@functools.partial(
    pl.kernel,
    mesh=_MESH,
    out_type=[
        jax.ShapeDtypeStruct((OUT_WORDS,), jnp.int32),
        jax.ShapeDtypeStruct((OUT_WORDS,), jnp.float32),
        jax.ShapeDtypeStruct((OUT_WORDS,), jnp.float32),
    ],
    scratch_types=[
        pltpu.VMEM((SPW,), jnp.int32),
        pltpu.VMEM((SPW * 2,), jnp.int32),
        pltpu.VMEM((SPW * 3,), jnp.int32),
        pltpu.VMEM((SPW * 3,), jnp.float32),
        pltpu.VMEM((SPW * 3,), jnp.float32),
        pltpu.VMEM((QWORDS,), jnp.float32),
        pltpu.VMEM((QWORDS,), jnp.int32),
        pltpu.SemaphoreType.DMA,
        pltpu.SemaphoreType.DMA,
    ],
    compiler_params=pltpu.CompilerParams(needs_layout_passes=False),
)
def _sample_gather(idx_hbm, dirs_hbm, orig_hbm,
                   coords_out, dirs_out, orig_out,
                   idx_v, coords_v, fidx_v, dgat_v, ogat_v, stage_f, stage_i,
                   dsem, osem):
    wid = lax.axis_index("s") * NC + lax.axis_index("c")
    b = wid // WPB
    pltpu.sync_copy(idx_hbm.at[pl.ds(wid * SPW, SPW)], idx_v)

    iota = lax.iota(jnp.int32, 16)
    tile_off = ((iota >> 3) << 10) + ((iota & 7) << 7)
    obase = b * OPLANE + (wid % WPB) * (SPW // 8) * 1024

    def build(j, carry):
        p0 = j * 16
        v = idx_v[pl.ds(p0, 16)]
        y = v >> 9
        x = v & 511
        coords_v[pl.ds(p0, 16)] = y
        coords_v[pl.ds(SPW + p0, 16)] = x
        toff = (((y >> 3) << 12) + ((x >> 7) << 10)
                + ((y & 7) << 7) + (x & 127))
        for c in range(3):
            fidx_v[pl.ds(c * SPW + p0, 16)] = toff + ((b * 3 + c) * HW)
        return carry

    lax.fori_loop(0, CHUNKS, build, 0)

    dcp = pltpu.async_copy(dirs_hbm.at[fidx_v], dgat_v, dsem)
    ocp = pltpu.async_copy(orig_hbm.at[fidx_v], ogat_v, osem)

    def emit(src_v, nch, stage, out_ref):
        for q in range(4):
            def fill(k, carry):
                dst = k * 2048 + tile_off
                for c in range(nch):
                    vals = src_v[pl.ds(c * SPW + q * QSAMP + k * 16, 16)]
                    plsc.store_scatter(stage, [dst + c], vals)
                return carry
            lax.fori_loop(0, QSAMP // 16, fill, 0)
            pltpu.sync_copy(stage, out_ref.at[pl.ds(obase + q * QWORDS, QWORDS)])

    emit(coords_v, 2, stage_i, coords_out)
    dcp.wait()
    emit(dgat_v, 3, stage_f, dirs_out)
    ocp.wait()
    emit(ogat_v, 3, stage_f, orig_out)


def kernel(n_sample, rays_directions, rays_origins):
    indices = jax.random.randint(jax.random.key(42), (B, N), 0, HW)
    idx_flat = indices.reshape(-1).astype(jnp.int32)

    def tiled_flat(a):
        return (a.reshape(B, 3, H // 8, 8, W // 128, 128)
                 .transpose(0, 1, 2, 4, 3, 5)
                 .reshape(-1))

    coords_pad, dirs_pad, orig_pad = _sample_gather(
        idx_flat,
        tiled_flat(rays_directions),
        tiled_flat(rays_origins),
    )

    sample_coordinates = coords_pad.reshape(B, N, 128)[:, :, :2]
    sampled_dirs = dirs_pad.reshape(B, N, 128)[:, :, :3]
    sampled_origins = orig_pad.reshape(B, N, 128)[:, :, :3]
    indices = indices + (jnp.asarray(n_sample, dtype=indices.dtype) * 0)
    return indices, sample_coordinates, sampled_dirs, sampled_origins


# speedup vs baseline: 5.0086x; 1.2179x over previous
import functools

import jax
import jax.numpy as jnp
from jax import lax
from jax.experimental import pallas as pl
from jax.experimental.pallas import tpu as pltpu
from jax.experimental.pallas import tpu_sc as plsc

H = 512
W = 512
B = 8
N = 4096
HW = H * W

NC = 2
NS = 16
NW = NC * NS
SPW = (B * N) // NW
CHUNKS = SPW // 16
WPB = NW // B

OPLANE = (N // 8) * 8 * 128
OUT_WORDS = B * OPLANE

QSAMP = SPW // 4
QWORDS = (QSAMP // 8) * 1024

_MESH = plsc.VectorSubcoreMesh(core_axis_name="c", subcore_axis_name="s")


@functools.partial(
    pl.kernel,
    mesh=_MESH,
    out_type=[
        jax.ShapeDtypeStruct((OUT_WORDS,), jnp.float32),
        jax.ShapeDtypeStruct((OUT_WORDS,), jnp.float32),
    ],
    scratch_types=[
        pltpu.VMEM((SPW,), jnp.int32),
        pltpu.VMEM((SPW * 3,), jnp.int32),
        pltpu.VMEM((SPW * 3,), jnp.float32),
        pltpu.VMEM((SPW * 3,), jnp.float32),
        pltpu.VMEM((QWORDS,), jnp.float32),
        pltpu.SemaphoreType.DMA,
        pltpu.SemaphoreType.DMA,
    ],
    compiler_params=pltpu.CompilerParams(needs_layout_passes=False),
)
def _sample_gather(idx_hbm, dirs_hbm, orig_hbm,
                   dirs_out, orig_out,
                   idx_v, fidx_v, dgat_v, ogat_v, stage_f,
                   dsem, osem):
    wid = lax.axis_index("s") * NC + lax.axis_index("c")
    b = wid // WPB
    pltpu.sync_copy(idx_hbm.at[pl.ds(wid * SPW, SPW)], idx_v)

    iota = lax.iota(jnp.int32, 16)
    tile_off = ((iota >> 3) << 10) + ((iota & 7) << 7)
    obase = b * OPLANE + (wid % WPB) * (SPW // 8) * 1024

    def build(j, carry):
        p0 = j * 16
        v = idx_v[pl.ds(p0, 16)]
        y = v >> 9
        x = v & 511
        toff = (((y >> 3) << 12) + ((x >> 7) << 10)
                + ((y & 7) << 7) + (x & 127))
        for c in range(3):
            fidx_v[pl.ds(c * SPW + p0, 16)] = toff + ((b * 3 + c) * HW)
        return carry

    lax.fori_loop(0, CHUNKS, build, 0)

    dcp = pltpu.async_copy(dirs_hbm.at[fidx_v], dgat_v, dsem)
    ocp = pltpu.async_copy(orig_hbm.at[fidx_v], ogat_v, osem)

    def emit(src_v, nch, stage, out_ref):
        for q in range(4):
            def fill(k, carry):
                dst = k * 2048 + tile_off
                for c in range(nch):
                    vals = src_v[pl.ds(c * SPW + q * QSAMP + k * 16, 16)]
                    plsc.store_scatter(stage, [dst + c], vals)
                return carry
            lax.fori_loop(0, QSAMP // 16, fill, 0)
            pltpu.sync_copy(stage, out_ref.at[pl.ds(obase + q * QWORDS, QWORDS)])

    dcp.wait()
    emit(dgat_v, 3, stage_f, dirs_out)
    ocp.wait()
    emit(ogat_v, 3, stage_f, orig_out)


def kernel(n_sample, rays_directions, rays_origins):
    indices = jax.random.randint(jax.random.key(42), (B, N), 0, HW)
    idx_flat = indices.reshape(-1).astype(jnp.int32)

    def tiled_flat(a):
        return (a.reshape(B, 3, H // 8, 8, W // 128, 128)
                 .transpose(0, 1, 2, 4, 3, 5)
                 .reshape(-1))

    dirs_pad, orig_pad = _sample_gather(
        idx_flat,
        tiled_flat(rays_directions),
        tiled_flat(rays_origins),
    )

    sample_coordinates = jnp.stack((indices >> 9, indices & 511), axis=-1)
    sample_coordinates = sample_coordinates.astype(jnp.int32)

    sampled_dirs = dirs_pad.reshape(B, N, 128)[:, :, :3]
    sampled_origins = orig_pad.reshape(B, N, 128)[:, :, :3]
    indices = indices + (jnp.asarray(n_sample, dtype=indices.dtype) * 0)
    return indices, sample_coordinates, sampled_dirs, sampled_origins


# speedup vs baseline: 5.0552x; 1.0093x over previous
import functools

import jax
import jax.numpy as jnp
from jax import lax
from jax.experimental import pallas as pl
from jax.experimental.pallas import tpu as pltpu
from jax.experimental.pallas import tpu_sc as plsc

H = 512
W = 512
B = 8
N = 4096
HW = H * W

NC = 2
NS = 16
NW = NC * NS
SPW = (B * N) // NW
CHUNKS = SPW // 16
WPB = NW // B

OPLANE = (N // 8) * 8 * 128
OUT_WORDS = B * OPLANE

QSAMP = SPW // 4
QWORDS = (QSAMP // 8) * 1024

_MESH = plsc.VectorSubcoreMesh(core_axis_name="c", subcore_axis_name="s")


@functools.partial(
    pl.kernel,
    mesh=_MESH,
    out_type=[
        jax.ShapeDtypeStruct((B, N, 3), jnp.float32),
        jax.ShapeDtypeStruct((B, N, 3), jnp.float32),
    ],
    scratch_types=[
        pltpu.VMEM((SPW,), jnp.int32),
        pltpu.VMEM((SPW * 3,), jnp.int32),
        pltpu.VMEM((SPW * 3,), jnp.float32),
        pltpu.VMEM((SPW * 3,), jnp.float32),
        pltpu.VMEM((512, 3), jnp.float32),
        pltpu.SemaphoreType.DMA,
        pltpu.SemaphoreType.DMA,
    ],
    compiler_params=pltpu.CompilerParams(needs_layout_passes=False),
)
def _sample_gather(idx_hbm, dirs_hbm, orig_hbm,
                   dirs_out, orig_out,
                   idx_v, fidx_v, dgat_v, ogat_v, stage_f,
                   dsem, osem):
    wid = lax.axis_index("s") * NC + lax.axis_index("c")
    b = wid // WPB
    pltpu.sync_copy(idx_hbm.at[pl.ds(wid * SPW, SPW)], idx_v)

    iota = lax.iota(jnp.int32, 16)
    n0 = (wid % WPB) * SPW

    def build(j, carry):
        p0 = j * 16
        v = idx_v[pl.ds(p0, 16)]
        y = v >> 9
        x = v & 511
        toff = (((y >> 3) << 12) + ((x >> 7) << 10)
                + ((y & 7) << 7) + (x & 127))
        for c in range(3):
            fidx_v[pl.ds(c * SPW + p0, 16)] = toff + ((b * 3 + c) * HW)
        return carry

    lax.fori_loop(0, CHUNKS, build, 0)

    dcp = pltpu.async_copy(dirs_hbm.at[fidx_v], dgat_v, dsem)
    ocp = pltpu.async_copy(orig_hbm.at[fidx_v], ogat_v, osem)

    def emit(src_v, nch, stage, out_ref):
        for h in range(2):
            def fill(k, carry):
                rows = k * 16 + iota
                for c in range(nch):
                    vals = src_v[pl.ds(c * SPW + h * 512 + k * 16, 16)]
                    plsc.store_scatter(stage, [rows, jnp.full((16,), c, jnp.int32)], vals)
                return carry
            lax.fori_loop(0, 512 // 16, fill, 0)
            pltpu.sync_copy(stage, out_ref.at[b, pl.ds(n0 + h * 512, 512)])

    dcp.wait()
    emit(dgat_v, 3, stage_f, dirs_out)
    ocp.wait()
    emit(ogat_v, 3, stage_f, orig_out)


def kernel(n_sample, rays_directions, rays_origins):
    indices = jax.random.randint(jax.random.key(42), (B, N), 0, HW)
    idx_flat = indices.reshape(-1).astype(jnp.int32)

    def tiled_flat(a):
        return (a.reshape(B, 3, H // 8, 8, W // 128, 128)
                 .transpose(0, 1, 2, 4, 3, 5)
                 .reshape(-1))

    dirs_pad, orig_pad = _sample_gather(
        idx_flat,
        tiled_flat(rays_directions),
        tiled_flat(rays_origins),
    )

    sample_coordinates = jnp.stack((indices >> 9, indices & 511), axis=-1)
    sample_coordinates = sample_coordinates.astype(jnp.int32)

    sampled_dirs = dirs_pad
    sampled_origins = orig_pad
    indices = indices + (jnp.asarray(n_sample, dtype=indices.dtype) * 0)
    return indices, sample_coordinates, sampled_dirs, sampled_origins


# speedup vs baseline: 5.2130x; 1.0312x over previous
import functools

import jax
import jax.numpy as jnp
from jax import lax
from jax.experimental import pallas as pl
from jax.experimental.pallas import tpu as pltpu
from jax.experimental.pallas import tpu_sc as plsc

H = 512
W = 512
B = 8
N = 4096
HW = H * W

NC = 2
NS = 16
NW = NC * NS
SPW = (B * N) // NW
CHUNKS = SPW // 16
WPB = NW // B

OPLANE = (N // 8) * 8 * 128
OUT_WORDS = B * OPLANE

QSAMP = SPW // 4
QWORDS = (QSAMP // 8) * 1024

_MESH = plsc.VectorSubcoreMesh(core_axis_name="c", subcore_axis_name="s")


@functools.partial(
    pl.kernel,
    mesh=_MESH,
    out_type=jax.ShapeDtypeStruct((B, N, 3), jnp.float32),
    scratch_types=[
        pltpu.VMEM((SPW,), jnp.int32),
        pltpu.VMEM((SPW * 3,), jnp.int32),
        pltpu.VMEM((SPW * 3,), jnp.float32),
        pltpu.VMEM((512, 3), jnp.float32),
        pltpu.SemaphoreType.DMA,
    ],
    compiler_params=pltpu.CompilerParams(needs_layout_passes=False),
)
def _sample_gather(idx_hbm, table_hbm,
                   out_ref,
                   idx_v, fidx_v, gat_v, stage, sem):
    wid = lax.axis_index("s") * NC + lax.axis_index("c")
    b = wid // WPB
    pltpu.sync_copy(idx_hbm.at[pl.ds(wid * SPW, SPW)], idx_v)

    iota = lax.iota(jnp.int32, 16)
    n0 = (wid % WPB) * SPW

    def build(j, carry):
        p0 = j * 16
        v = idx_v[pl.ds(p0, 16)]
        y = v >> 9
        x = v & 511
        toff = (((y >> 3) << 12) + ((x >> 7) << 10)
                + ((y & 7) << 7) + (x & 127))
        for c in range(3):
            fidx_v[pl.ds(c * SPW + p0, 16)] = toff + ((b * 3 + c) * HW)
        return carry

    lax.fori_loop(0, CHUNKS, build, 0)

    pltpu.async_copy(table_hbm.at[fidx_v], gat_v, sem).wait()

    for h in range(2):
        def fill(k, carry):
            rows = k * 16 + iota
            for c in range(3):
                vals = gat_v[pl.ds(c * SPW + h * 512 + k * 16, 16)]
                plsc.store_scatter(stage, [rows, jnp.full((16,), c, jnp.int32)], vals)
            return carry
        lax.fori_loop(0, 512 // 16, fill, 0)
        pltpu.sync_copy(stage, out_ref.at[b, pl.ds(n0 + h * 512, 512)])


def kernel(n_sample, rays_directions, rays_origins):
    indices = jax.random.randint(jax.random.key(42), (B, N), 0, HW)
    idx_flat = indices.reshape(-1).astype(jnp.int32)

    def tiled_flat(a):
        return (a.reshape(B, 3, H // 8, 8, W // 128, 128)
                 .transpose(0, 1, 2, 4, 3, 5)
                 .reshape(-1))

    sampled_dirs = _sample_gather(idx_flat, tiled_flat(rays_directions))
    sampled_origins = _sample_gather(idx_flat, tiled_flat(rays_origins))

    sample_coordinates = jnp.stack((indices >> 9, indices & 511), axis=-1)
    sample_coordinates = sample_coordinates.astype(jnp.int32)

    indices = indices + (jnp.asarray(n_sample, dtype=indices.dtype) * 0)
    return indices, sample_coordinates, sampled_dirs, sampled_origins
